# Initial kernel scaffold; baseline (speedup 1.0000x reference)
import jax
import jax.numpy as jnp
from jax.experimental import pallas as pl


def kernel(x, edge_index, W1, att_src1, att_dst1, b1, W2, att_src2, att_dst2, b2, W3, att_src3, att_dst3, b3, W4, att_src4, att_dst4, b4, fc_W, fc_b):
    raise NotImplementedError("write your pallas kernel here")


# speedup vs baseline: 8.6851x; 8.6851x over previous
import jax
import jax.numpy as jnp
from jax import lax
from jax.experimental import pallas as pl
from jax.experimental.pallas import tpu as pltpu
from jax.experimental.pallas import tpu_sc as plsc

N_NODES = 10000
N_EDGES = 320000 + N_NODES
NP = 10240
CHUNK = 128
N_WORKERS = 32
ROWS_PER_TILE = 88
E_PAD = N_WORKERS * ROWS_PER_TILE * CHUNK
NEG_BIG = -1e30
_BLK = 256
_GRID = NP // _BLK


def _pad2(a, r, c):
    return jnp.pad(a, ((0, r - a.shape[0]), (0, c - a.shape[1])))


def _ones_col(h, col):
    ci = lax.broadcasted_iota(jnp.int32, h.shape, 1)
    return h + (ci == col).astype(jnp.float32)


def _aux(v2, g, aux_ref):
    a = lax.dot_general(v2, g, (((1,), (1,)), ((), ())),
                        preferred_element_type=jnp.float32)
    t = a[0:1, :] + a[8:9, :]
    ad2 = jnp.where(t >= 0, t, 0.2 * t)
    aux_ref[0:16, :] = a
    aux_ref[16:24, :] = jnp.broadcast_to(ad2, (8, a.shape[1]))


def _norm(a, ic, b):
    s = a[:, ic:ic + 1]
    s = jnp.where(s > 0, s, 1.0)
    return jnp.maximum(a / s + b, 0.0)


def _tc_first_body(x_ref, w_ref, v2_ref, h_ref, aux_ref):
    g = x_ref[...]
    h = jnp.dot(g, w_ref[...], preferred_element_type=jnp.float32)
    h_ref[...] = _ones_col(h, 32)
    _aux(v2_ref[...], g, aux_ref)


def _tc_mid1_body(acc_ref, w_ref, v2_ref, b_ref, h_ref, aux_ref):
    g = _norm(acc_ref[0] + acc_ref[1], acc_ref.shape[2] - 16, b_ref[...])
    h = jnp.dot(g, w_ref[...], preferred_element_type=jnp.float32)
    h_ref[...] = _ones_col(h, w_ref.shape[1] - 16)
    _aux(v2_ref[...], g, aux_ref)


def _tc_mid_s_body(acc_ref, wa_ref, wb_ref, v2_ref, b_ref,
                   h1_ref, h2_ref, aux_ref):
    g = _norm(acc_ref[0] + acc_ref[1], acc_ref.shape[2] - 16, b_ref[...])
    h1 = jnp.dot(g, wa_ref[...], preferred_element_type=jnp.float32)
    h1_ref[...] = _ones_col(h1, 64)
    h2_ref[...] = jnp.dot(g, wb_ref[...], preferred_element_type=jnp.float32)
    _aux(v2_ref[...], g, aux_ref)


def _tc_mid_ss_body(a1_ref, a2_ref, waa_ref, wab_ref, wba_ref, wbb_ref,
                    v2a_ref, v2b_ref, ba_ref, bb_ref, h1_ref, h2_ref, aux_ref):
    a1 = a1_ref[0] + a1_ref[1]
    a2 = a2_ref[0] + a2_ref[1]
    s = a1[:, 64:65]
    s = jnp.where(s > 0, s, 1.0)
    g1 = jnp.maximum(a1 / s + ba_ref[...], 0.0)
    g2 = jnp.maximum(a2 / s + bb_ref[...], 0.0)
    h1 = (jnp.dot(g1, waa_ref[...], preferred_element_type=jnp.float32)
          + jnp.dot(g2, wba_ref[...], preferred_element_type=jnp.float32))
    h1_ref[...] = _ones_col(h1, 64)
    h2_ref[...] = (jnp.dot(g1, wab_ref[...], preferred_element_type=jnp.float32)
                   + jnp.dot(g2, wbb_ref[...], preferred_element_type=jnp.float32))
    a = (lax.dot_general(v2a_ref[...], g1, (((1,), (1,)), ((), ())),
                         preferred_element_type=jnp.float32)
         + lax.dot_general(v2b_ref[...], g2, (((1,), (1,)), ((), ())),
                           preferred_element_type=jnp.float32))
    t = a[0:1, :] + a[8:9, :]
    ad2 = jnp.where(t >= 0, t, 0.2 * t)
    aux_ref[0:16, :] = a
    aux_ref[16:24, :] = jnp.broadcast_to(ad2, (8, a.shape[1]))


def _tc_last_body(a1_ref, a2_ref, wa_ref, wb_ref, ba_ref, bb_ref, fcb_ref,
                  out_ref):
    a1 = a1_ref[0] + a1_ref[1]
    a2 = a2_ref[0] + a2_ref[1]
    s = a1[:, 64:65]
    s = jnp.where(s > 0, s, 1.0)
    g1 = jnp.maximum(a1 / s + ba_ref[...], 0.0)
    g2 = jnp.maximum(a2 / s + bb_ref[...], 0.0)
    out_ref[...] = (jnp.dot(g1, wa_ref[...], preferred_element_type=jnp.float32)
                    + jnp.dot(g2, wb_ref[...], preferred_element_type=jnp.float32)
                    + fcb_ref[...])


def _row_spec(shape):
    nd = len(shape)
    if nd == 2:
        return pl.BlockSpec((_BLK, shape[1]), lambda i: (i, 0))
    return pl.BlockSpec((2, _BLK, shape[2]), lambda i: (0, i, 0))


def _full_spec(shape):
    nd = len(shape)
    return pl.BlockSpec(shape, lambda i: (0,) * nd)


def _tc_call(body, row_in, full_in, row_out_shapes, aux_out=True):
    in_specs = [_row_spec(a.shape) for a in row_in] + \
               [_full_spec(a.shape) for a in full_in]
    out_specs = [_row_spec(s) for s in row_out_shapes]
    out_shape = [jax.ShapeDtypeStruct(s, jnp.float32) for s in row_out_shapes]
    if aux_out:
        out_specs.append(pl.BlockSpec((24, _BLK), lambda i: (0, i)))
        out_shape.append(jax.ShapeDtypeStruct((24, NP), jnp.float32))
    return pl.pallas_call(
        body, grid=(_GRID,), in_specs=in_specs,
        out_specs=out_specs if len(out_specs) > 1 else out_specs[0],
        out_shape=out_shape if len(out_shape) > 1 else out_shape[0],
    )(*row_in, *full_in)


def _make_sc_edge(widths, interpret=False):
    nt = len(widths)
    rows_per_sub = NP // 16

    def body(*refs):
        hs = refs[:nt]
        aux_hbm, src_hbm, dst_hbm = refs[nt:nt + 3]
        outs = refs[nt + 3:nt + 3 + nt]
        sc = refs[nt + 3 + nt:]
        srcv, dstv, asv, adv, ad2v, wbuf = sc[:6]
        rowbufs = sc[6:6 + nt]
        accs = sc[6 + nt:]

        c = lax.axis_index("c")
        s = lax.axis_index("s")
        wid = c * 16 + s

        zero = jnp.zeros((16,), jnp.float32)

        def zr(e, carry):
            for t in range(nt):
                for j in range(widths[t] // 16):
                    rowbufs[t][e, pl.ds(j * 16, 16)] = zero
            return carry

        lax.fori_loop(0, CHUNK, zr, 0)
        for k in range(rows_per_sub // CHUNK):
            for t in range(nt):
                pltpu.sync_copy(
                    rowbufs[t],
                    accs[t].at[pl.ds(s * rows_per_sub + k * CHUNK, CHUNK)])

        pltpu.sync_copy(aux_hbm.at[0], asv)
        pltpu.sync_copy(aux_hbm.at[8], adv)
        pltpu.sync_copy(aux_hbm.at[16], ad2v)
        pltpu.sync_copy(src_hbm.at[pl.ds(wid * ROWS_PER_TILE, ROWS_PER_TILE)],
                        srcv)
        pltpu.sync_copy(dst_hbm.at[pl.ds(wid * ROWS_PER_TILE, ROWS_PER_TILE)],
                        dstv)

        plsc.subcore_barrier()

        def chunk(j, carry):
            for t in range(nt):
                pltpu.sync_copy(hs[t].at[srcv.at[j]], rowbufs[t])
            for g in range(CHUNK // 16):
                sv = srcv[j, pl.ds(g * 16, 16)]
                dv = dstv[j, pl.ds(g * 16, 16)]
                av = plsc.load_gather(asv, [sv])
                bv = plsc.load_gather(adv, [dv])
                cv = plsc.load_gather(ad2v, [dv])
                xl = av + bv
                l1 = jnp.where(xl >= 0, xl, 0.2 * xl)
                wbuf[pl.ds(g * 16, 16)] = jnp.exp(l1 - cv)

            def scale(e, carry2):
                wsp = plsc.load_gather(wbuf, [jnp.zeros((16,), jnp.int32) + e])
                for t in range(nt):
                    for jj in range(widths[t] // 16):
                        rowbufs[t][e, pl.ds(jj * 16, 16)] = \
                            rowbufs[t][e, pl.ds(jj * 16, 16)] * wsp
                return carry2

            lax.fori_loop(0, CHUNK, scale, 0)
            for t in range(nt):
                pltpu.sync_copy(rowbufs[t], accs[t].at[dstv.at[j]], add=True)
            return carry

        lax.fori_loop(0, ROWS_PER_TILE, chunk, 0)
        plsc.subcore_barrier()

        for k in range(rows_per_sub // CHUNK):
            off = s * rows_per_sub + k * CHUNK
            for t in range(nt):
                pltpu.sync_copy(accs[t].at[pl.ds(off, CHUNK)], rowbufs[t])
                pltpu.sync_copy(rowbufs[t], outs[t].at[c].at[pl.ds(off, CHUNK)])

    mesh = plsc.VectorSubcoreMesh(core_axis_name="c", subcore_axis_name="s",
                                  num_cores=2, num_subcores=16)
    out_type = tuple(jax.ShapeDtypeStruct((2, NP, w), jnp.float32)
                     for w in widths)
    scratch = [
        pltpu.VMEM((ROWS_PER_TILE, CHUNK), jnp.int32),
        pltpu.VMEM((ROWS_PER_TILE, CHUNK), jnp.int32),
        pltpu.VMEM((NP,), jnp.float32),
        pltpu.VMEM((NP,), jnp.float32),
        pltpu.VMEM((NP,), jnp.float32),
        pltpu.VMEM((CHUNK,), jnp.float32),
    ] + [pltpu.VMEM((CHUNK, w), jnp.float32) for w in widths] \
      + [pltpu.VMEM_SHARED((NP, w), jnp.float32) for w in widths]
    return pl.kernel(body, out_type=out_type, mesh=mesh,
                     scratch_types=scratch, interpret=interpret,
                     compiler_params=pltpu.CompilerParams(
                         needs_layout_passes=False,
                         use_tc_tiling_on_sc=False))


_SC_EDGE = {ws: _make_sc_edge(ws) for ws in ((48,), (80,), (64,))}


def kernel(x, edge_index, W1, att_src1, att_dst1, b1, W2, att_src2, att_dst2,
           b2, W3, att_src3, att_dst3, b3, W4, att_src4, att_dst4, b4,
           fc_W, fc_b):
    f32 = jnp.float32
    loops = jnp.arange(N_NODES, dtype=jnp.int32)
    pad_n = E_PAD - N_EDGES
    src = jnp.concatenate([edge_index[0], loops,
                           jnp.zeros((pad_n,), jnp.int32)]).reshape(-1, CHUNK)
    dst = jnp.concatenate([edge_index[1], loops,
                           jnp.full((pad_n,), N_NODES, jnp.int32)]).reshape(-1, CHUNK)
    x_pad = _pad2(x, NP, 128)

    def v2_of(v, width):
        out = jnp.zeros((16, width), f32)
        return out.at[0, :v[0].shape[0]].set(v[0]).at[8, :v[1].shape[0]].set(v[1])

    w1p = _pad2(W1, 128, 48)
    v2 = v2_of((W1 @ att_src1, W1 @ att_dst1), 128)
    h, aux = _tc_call(_tc_first_body, [x_pad], [w1p, v2],
                      [(NP, 48)])
    acc1, = _SC_EDGE[(48,)](h, aux, src, dst)

    w2p = _pad2(W2, 48, 80)
    v2 = v2_of((W2 @ att_src2, W2 @ att_dst2), 48)
    bp = jnp.zeros((1, 48), f32).at[0, :32].set(b1)
    h, aux = _tc_call(_tc_mid1_body, [acc1], [w2p, v2, bp], [(NP, 80)])
    acc2, = _SC_EDGE[(80,)](h, aux, src, dst)

    w3a = _pad2(W3[:, :64], 80, 80)
    w3b = _pad2(W3[:, 64:], 80, 64)
    v2 = v2_of((W3 @ att_src3, W3 @ att_dst3), 80)
    bp = jnp.zeros((1, 80), f32).at[0, :64].set(b2)
    h1, h2, aux = _tc_call(_tc_mid_s_body, [acc2], [w3a, w3b, v2, bp],
                           [(NP, 80), (NP, 64)])
    acc3a, = _SC_EDGE[(80,)](h1, aux, src, dst)
    acc3b, = _SC_EDGE[(64,)](h2, aux, src, dst)

    w4aa = _pad2(W4[:64, :64], 80, 80)
    w4ab = _pad2(W4[:64, 64:], 80, 64)
    w4ba = _pad2(W4[64:, :64], 64, 80)
    w4bb = W4[64:, 64:]
    v2a = v2_of((W4[:64] @ att_src4, W4[:64] @ att_dst4), 80)
    v2b = v2_of((W4[64:] @ att_src4, W4[64:] @ att_dst4), 64)
    bpa = jnp.zeros((1, 80), f32).at[0, :64].set(b3[:64])
    bpb = b3[64:].reshape(1, 64)
    h1, h2, aux = _tc_call(_tc_mid_ss_body, [acc3a, acc3b],
                           [w4aa, w4ab, w4ba, w4bb, v2a, v2b, bpa, bpb],
                           [(NP, 80), (NP, 64)])
    acc4a, = _SC_EDGE[(80,)](h1, aux, src, dst)
    acc4b, = _SC_EDGE[(64,)](h2, aux, src, dst)

    fca = _pad2(fc_W[:64], 80, 128)
    fcb_w = fc_W[64:]
    bpa = jnp.zeros((1, 80), f32).at[0, :64].set(b4[:64])
    bpb = b4[64:].reshape(1, 64)
    out = _tc_call(_tc_last_body, [acc4a, acc4b],
                   [fca, fcb_w, bpa, bpb, fc_b.reshape(1, 128)],
                   [(NP, 128)], aux_out=False)
    return out[:N_NODES]


# speedup vs baseline: 21.7111x; 2.4998x over previous
import jax
import jax.numpy as jnp
from jax import lax
from jax.experimental import pallas as pl
from jax.experimental.pallas import tpu as pltpu
from jax.experimental.pallas import tpu_sc as plsc

N_NODES = 10000
N_EDGES = 320000 + N_NODES
NP = 10240
CHUNK = 128
N_WORKERS = 32
ROWS_PER_TILE = 82
E_PAD = N_WORKERS * ROWS_PER_TILE * CHUNK
NEG_BIG = -1e30
_BLK = 256
_GRID = NP // _BLK


def _pad2(a, r, c):
    return jnp.pad(a, ((0, r - a.shape[0]), (0, c - a.shape[1])))


def _ones_col(h, col):
    ci = lax.broadcasted_iota(jnp.int32, h.shape, 1)
    return h + (ci == col).astype(jnp.float32)


def _aux(v2, g, aux_ref):
    a = lax.dot_general(v2, g, (((1,), (1,)), ((), ())),
                        preferred_element_type=jnp.float32)
    t = a[0:1, :] + a[8:9, :]
    ad2 = jnp.where(t >= 0, t, 0.2 * t)
    aux_ref[0:16, :] = a
    aux_ref[16:24, :] = jnp.broadcast_to(ad2, (8, a.shape[1]))


def _norm(a, ic, b):
    s = a[:, ic:ic + 1]
    s = jnp.where(s > 0, s, 1.0)
    return jnp.maximum(a / s + b, 0.0)


def _tc_first_body(x_ref, w_ref, v2_ref, h_ref, aux_ref):
    g = x_ref[...]
    h = jnp.dot(g, w_ref[...], preferred_element_type=jnp.float32)
    h_ref[...] = _ones_col(h, 32)
    _aux(v2_ref[...], g, aux_ref)


def _tc_mid1_body(acc_ref, w_ref, v2_ref, b_ref, h_ref, aux_ref):
    g = _norm(acc_ref[0] + acc_ref[1], acc_ref.shape[2] - 16, b_ref[...])
    h = jnp.dot(g, w_ref[...], preferred_element_type=jnp.float32)
    h_ref[...] = _ones_col(h, w_ref.shape[1] - 16)
    _aux(v2_ref[...], g, aux_ref)


def _tc_mid_s_body(acc_ref, wa_ref, wb_ref, v2_ref, b_ref,
                   h1_ref, h2_ref, aux_ref):
    g = _norm(acc_ref[0] + acc_ref[1], acc_ref.shape[2] - 16, b_ref[...])
    h1 = jnp.dot(g, wa_ref[...], preferred_element_type=jnp.float32)
    h1_ref[...] = _ones_col(h1, 64)
    h2_ref[...] = jnp.dot(g, wb_ref[...], preferred_element_type=jnp.float32)
    _aux(v2_ref[...], g, aux_ref)


def _tc_mid_ss_body(a1_ref, a2_ref, waa_ref, wab_ref, wba_ref, wbb_ref,
                    v2a_ref, v2b_ref, ba_ref, bb_ref, h1_ref, h2_ref, aux_ref):
    a1 = a1_ref[0] + a1_ref[1]
    a2 = a2_ref[0] + a2_ref[1]
    s = a1[:, 64:65]
    s = jnp.where(s > 0, s, 1.0)
    g1 = jnp.maximum(a1 / s + ba_ref[...], 0.0)
    g2 = jnp.maximum(a2 / s + bb_ref[...], 0.0)
    h1 = (jnp.dot(g1, waa_ref[...], preferred_element_type=jnp.float32)
          + jnp.dot(g2, wba_ref[...], preferred_element_type=jnp.float32))
    h1_ref[...] = _ones_col(h1, 64)
    h2_ref[...] = (jnp.dot(g1, wab_ref[...], preferred_element_type=jnp.float32)
                   + jnp.dot(g2, wbb_ref[...], preferred_element_type=jnp.float32))
    a = (lax.dot_general(v2a_ref[...], g1, (((1,), (1,)), ((), ())),
                         preferred_element_type=jnp.float32)
         + lax.dot_general(v2b_ref[...], g2, (((1,), (1,)), ((), ())),
                           preferred_element_type=jnp.float32))
    t = a[0:1, :] + a[8:9, :]
    ad2 = jnp.where(t >= 0, t, 0.2 * t)
    aux_ref[0:16, :] = a
    aux_ref[16:24, :] = jnp.broadcast_to(ad2, (8, a.shape[1]))


def _tc_last_body(a1_ref, a2_ref, wa_ref, wb_ref, ba_ref, bb_ref, fcb_ref,
                  out_ref):
    a1 = a1_ref[0] + a1_ref[1]
    a2 = a2_ref[0] + a2_ref[1]
    s = a1[:, 64:65]
    s = jnp.where(s > 0, s, 1.0)
    g1 = jnp.maximum(a1 / s + ba_ref[...], 0.0)
    g2 = jnp.maximum(a2 / s + bb_ref[...], 0.0)
    out_ref[...] = (jnp.dot(g1, wa_ref[...], preferred_element_type=jnp.float32)
                    + jnp.dot(g2, wb_ref[...], preferred_element_type=jnp.float32)
                    + fcb_ref[...])


def _row_spec(shape):
    nd = len(shape)
    if nd == 2:
        return pl.BlockSpec((_BLK, shape[1]), lambda i: (i, 0))
    return pl.BlockSpec((2, _BLK, shape[2]), lambda i: (0, i, 0))


def _full_spec(shape):
    nd = len(shape)
    return pl.BlockSpec(shape, lambda i: (0,) * nd)


def _tc_call(body, row_in, full_in, row_out_shapes, aux_out=True):
    in_specs = [_row_spec(a.shape) for a in row_in] + \
               [_full_spec(a.shape) for a in full_in]
    out_specs = [_row_spec(s) for s in row_out_shapes]
    out_shape = [jax.ShapeDtypeStruct(s, jnp.float32) for s in row_out_shapes]
    if aux_out:
        out_specs.append(pl.BlockSpec((24, _BLK), lambda i: (0, i)))
        out_shape.append(jax.ShapeDtypeStruct((24, NP), jnp.float32))
    return pl.pallas_call(
        body, grid=(_GRID,), in_specs=in_specs,
        out_specs=out_specs if len(out_specs) > 1 else out_specs[0],
        out_shape=out_shape if len(out_shape) > 1 else out_shape[0],
    )(*row_in, *full_in)


def _make_sc_edge(width, interpret=False):
    nvec = width // 16
    rows_per_sub = NP // 16

    def body(h_hbm, aux_hbm, src_hbm, dst_hbm, out_hbm,
             srcv, dstv, asv, adv, ad2v, wbuf, rows_a, rows_b, acc,
             sem_a, sem_b):
        c = lax.axis_index("c")
        s = lax.axis_index("s")
        wid = s * 2 + c

        zero = jnp.zeros((16,), jnp.float32)

        def zr(e, carry):
            for j in range(nvec):
                rows_a[e, pl.ds(j * 16, 16)] = zero
            return carry

        lax.fori_loop(0, CHUNK, zr, 0)
        for k in range(rows_per_sub // CHUNK):
            pltpu.sync_copy(rows_a,
                            acc.at[pl.ds(s * rows_per_sub + k * CHUNK, CHUNK)])

        pltpu.sync_copy(aux_hbm.at[0], asv)
        pltpu.sync_copy(aux_hbm.at[8], adv)
        pltpu.sync_copy(aux_hbm.at[16], ad2v)
        pltpu.sync_copy(src_hbm.at[pl.ds(wid * ROWS_PER_TILE, ROWS_PER_TILE)],
                        srcv)
        pltpu.sync_copy(dst_hbm.at[pl.ds(wid * ROWS_PER_TILE, ROWS_PER_TILE)],
                        dstv)

        plsc.subcore_barrier()

        def gat(j, buf, sem):
            return pltpu.make_async_copy(h_hbm.at[srcv.at[j]], buf, sem)

        def compute_w(j):
            for g in range(CHUNK // 16):
                sv = srcv[j, pl.ds(g * 16, 16)]
                dv = dstv[j, pl.ds(g * 16, 16)]
                av = plsc.load_gather(asv, [sv])
                bv = plsc.load_gather(adv, [dv])
                cv = plsc.load_gather(ad2v, [dv])
                xl = av + bv
                l1 = jnp.where(xl >= 0, xl, 0.2 * xl)
                wbuf[pl.ds(g * 16, 16)] = jnp.exp(l1 - cv)

        def scale(buf):
            def sc4(e4, carry2):
                for u in range(4):
                    e = e4 * 4 + u
                    wsp = plsc.load_gather(
                        wbuf, [jnp.zeros((16,), jnp.int32) + e])
                    for jj in range(nvec):
                        buf[e, pl.ds(jj * 16, 16)] = \
                            buf[e, pl.ds(jj * 16, 16)] * wsp
                return carry2

            lax.fori_loop(0, CHUNK // 4, sc4, 0)

        gat(0, rows_a, sem_a).start()

        def pair(k, carry):
            j0 = 2 * k
            j1 = j0 + 1
            j2 = jnp.minimum(j0 + 2, ROWS_PER_TILE - 1)
            gat(j0, rows_a, sem_a).wait()
            gat(j1, rows_b, sem_b).start()
            compute_w(j0)
            scale(rows_a)
            pltpu.sync_copy(rows_a, acc.at[dstv.at[j0]], add=True)
            gat(j1, rows_b, sem_b).wait()
            gat(j2, rows_a, sem_a).start()
            compute_w(j1)
            scale(rows_b)
            pltpu.sync_copy(rows_b, acc.at[dstv.at[j1]], add=True)
            return carry

        lax.fori_loop(0, ROWS_PER_TILE // 2, pair, 0)
        gat(ROWS_PER_TILE - 1, rows_a, sem_a).wait()
        plsc.subcore_barrier()

        for k in range(rows_per_sub // CHUNK):
            off = s * rows_per_sub + k * CHUNK
            pltpu.sync_copy(acc.at[pl.ds(off, CHUNK)], rows_a)
            pltpu.sync_copy(rows_a, out_hbm.at[c].at[pl.ds(off, CHUNK)])

    mesh = plsc.VectorSubcoreMesh(core_axis_name="c", subcore_axis_name="s",
                                  num_cores=2, num_subcores=16)
    scratch = [
        pltpu.VMEM((ROWS_PER_TILE, CHUNK), jnp.int32),
        pltpu.VMEM((ROWS_PER_TILE, CHUNK), jnp.int32),
        pltpu.VMEM((NP,), jnp.float32),
        pltpu.VMEM((NP,), jnp.float32),
        pltpu.VMEM((NP,), jnp.float32),
        pltpu.VMEM((CHUNK,), jnp.float32),
        pltpu.VMEM((CHUNK, width), jnp.float32),
        pltpu.VMEM((CHUNK, width), jnp.float32),
        pltpu.VMEM_SHARED((NP, width), jnp.float32),
        pltpu.SemaphoreType.DMA,
        pltpu.SemaphoreType.DMA,
    ]
    return pl.kernel(body,
                     out_type=jax.ShapeDtypeStruct((2, NP, width), jnp.float32),
                     mesh=mesh, scratch_types=scratch, interpret=interpret,
                     compiler_params=pltpu.CompilerParams(
                         needs_layout_passes=False,
                         use_tc_tiling_on_sc=False))


_SC_EDGE = {w: _make_sc_edge(w) for w in (48, 80, 64)}


def kernel(x, edge_index, W1, att_src1, att_dst1, b1, W2, att_src2, att_dst2,
           b2, W3, att_src3, att_dst3, b3, W4, att_src4, att_dst4, b4,
           fc_W, fc_b):
    f32 = jnp.float32
    loops = jnp.arange(N_NODES, dtype=jnp.int32)
    pad_n = E_PAD - N_EDGES
    src = jnp.concatenate([edge_index[0], loops,
                           jnp.zeros((pad_n,), jnp.int32)]).reshape(-1, CHUNK)
    dst = jnp.concatenate([edge_index[1], loops,
                           jnp.full((pad_n,), N_NODES, jnp.int32)]).reshape(-1, CHUNK)
    x_pad = _pad2(x, NP, 128)

    def v2_of(v, width):
        out = jnp.zeros((16, width), f32)
        return out.at[0, :v[0].shape[0]].set(v[0]).at[8, :v[1].shape[0]].set(v[1])

    w1p = _pad2(W1, 128, 48)
    v2 = v2_of((W1 @ att_src1, W1 @ att_dst1), 128)
    h, aux = _tc_call(_tc_first_body, [x_pad], [w1p, v2],
                      [(NP, 48)])
    acc1 = _SC_EDGE[48](h, aux, src, dst)

    w2p = _pad2(W2, 48, 80)
    v2 = v2_of((W2 @ att_src2, W2 @ att_dst2), 48)
    bp = jnp.zeros((1, 48), f32).at[0, :32].set(b1)
    h, aux = _tc_call(_tc_mid1_body, [acc1], [w2p, v2, bp], [(NP, 80)])
    acc2 = _SC_EDGE[80](h, aux, src, dst)

    w3a = _pad2(W3[:, :64], 80, 80)
    w3b = _pad2(W3[:, 64:], 80, 64)
    v2 = v2_of((W3 @ att_src3, W3 @ att_dst3), 80)
    bp = jnp.zeros((1, 80), f32).at[0, :64].set(b2)
    h1, h2, aux = _tc_call(_tc_mid_s_body, [acc2], [w3a, w3b, v2, bp],
                           [(NP, 80), (NP, 64)])
    acc3a = _SC_EDGE[80](h1, aux, src, dst)
    acc3b = _SC_EDGE[64](h2, aux, src, dst)

    w4aa = _pad2(W4[:64, :64], 80, 80)
    w4ab = _pad2(W4[:64, 64:], 80, 64)
    w4ba = _pad2(W4[64:, :64], 64, 80)
    w4bb = W4[64:, 64:]
    v2a = v2_of((W4[:64] @ att_src4, W4[:64] @ att_dst4), 80)
    v2b = v2_of((W4[64:] @ att_src4, W4[64:] @ att_dst4), 64)
    bpa = jnp.zeros((1, 80), f32).at[0, :64].set(b3[:64])
    bpb = b3[64:].reshape(1, 64)
    h1, h2, aux = _tc_call(_tc_mid_ss_body, [acc3a, acc3b],
                           [w4aa, w4ab, w4ba, w4bb, v2a, v2b, bpa, bpb],
                           [(NP, 80), (NP, 64)])
    acc4a = _SC_EDGE[80](h1, aux, src, dst)
    acc4b = _SC_EDGE[64](h2, aux, src, dst)

    fca = _pad2(fc_W[:64], 80, 128)
    fcb_w = fc_W[64:]
    bpa = jnp.zeros((1, 80), f32).at[0, :64].set(b4[:64])
    bpb = b4[64:].reshape(1, 64)
    out = _tc_call(_tc_last_body, [acc4a, acc4b],
                   [fca, fcb_w, bpa, bpb, fc_b.reshape(1, 128)],
                   [(NP, 128)], aux_out=False)
    return out[:N_NODES]


# speedup vs baseline: 32.7657x; 1.5092x over previous
import jax
import jax.numpy as jnp
from jax import lax
from jax.experimental import pallas as pl
from jax.experimental.pallas import tpu as pltpu
from jax.experimental.pallas import tpu_sc as plsc

N_NODES = 10000
N_EDGES = 320000 + N_NODES
NP = 10240
CHUNK = 96
N_WORKERS = 32
ROWS_PER_TILE = 108
E_PAD = N_WORKERS * ROWS_PER_TILE * CHUNK
NEG_BIG = -1e30
_BLK = 256
_GRID = NP // _BLK


def _pad2(a, r, c):
    return jnp.pad(a, ((0, r - a.shape[0]), (0, c - a.shape[1])))


def _ones_col(h, col):
    ci = lax.broadcasted_iota(jnp.int32, h.shape, 1)
    return h + (ci == col).astype(jnp.float32)


def _aux(v2, g, aux_ref):
    aux_ref[...] = lax.dot_general(v2, g, (((1,), (1,)), ((), ())),
                                   preferred_element_type=jnp.float32)


def _norm(a, ic, b):
    s = a[:, ic:ic + 1]
    s = jnp.where(s > 0, s, 1.0)
    return jnp.maximum(a / s + b, 0.0)


def _tc_first_body(x_ref, w_ref, v2_ref, h_ref, aux_ref):
    g = x_ref[...]
    h = jnp.dot(g, w_ref[...], preferred_element_type=jnp.float32)
    h_ref[...] = _ones_col(h, 32)
    _aux(v2_ref[...], g, aux_ref)


def _tc_mid1_body(acc_ref, w_ref, v2_ref, b_ref, h_ref, aux_ref):
    g = _norm(acc_ref[0] + acc_ref[1], acc_ref.shape[2] - 16, b_ref[...])
    h = jnp.dot(g, w_ref[...], preferred_element_type=jnp.float32)
    h_ref[...] = _ones_col(h, w_ref.shape[1] - 16)
    _aux(v2_ref[...], g, aux_ref)


def _tc_mid_s_body(acc_ref, wa_ref, wb_ref, v2_ref, b_ref,
                   h1_ref, h2_ref, aux_ref):
    g = _norm(acc_ref[0] + acc_ref[1], acc_ref.shape[2] - 16, b_ref[...])
    h1 = jnp.dot(g, wa_ref[...], preferred_element_type=jnp.float32)
    h1_ref[...] = _ones_col(h1, 64)
    h2_ref[...] = jnp.dot(g, wb_ref[...], preferred_element_type=jnp.float32)
    _aux(v2_ref[...], g, aux_ref)


def _tc_mid_ss_body(a1_ref, a2_ref, waa_ref, wab_ref, wba_ref, wbb_ref,
                    v2a_ref, v2b_ref, ba_ref, bb_ref, h1_ref, h2_ref, aux_ref):
    a1 = a1_ref[0] + a1_ref[1]
    a2 = a2_ref[0] + a2_ref[1]
    s = a1[:, 64:65]
    s = jnp.where(s > 0, s, 1.0)
    g1 = jnp.maximum(a1 / s + ba_ref[...], 0.0)
    g2 = jnp.maximum(a2 / s + bb_ref[...], 0.0)
    h1 = (jnp.dot(g1, waa_ref[...], preferred_element_type=jnp.float32)
          + jnp.dot(g2, wba_ref[...], preferred_element_type=jnp.float32))
    h1_ref[...] = _ones_col(h1, 64)
    h2_ref[...] = (jnp.dot(g1, wab_ref[...], preferred_element_type=jnp.float32)
                   + jnp.dot(g2, wbb_ref[...], preferred_element_type=jnp.float32))
    aux_ref[...] = (
        lax.dot_general(v2a_ref[...], g1, (((1,), (1,)), ((), ())),
                        preferred_element_type=jnp.float32)
        + lax.dot_general(v2b_ref[...], g2, (((1,), (1,)), ((), ())),
                          preferred_element_type=jnp.float32))


def _tc_last_body(a1_ref, a2_ref, wa_ref, wb_ref, ba_ref, bb_ref, fcb_ref,
                  out_ref):
    a1 = a1_ref[0] + a1_ref[1]
    a2 = a2_ref[0] + a2_ref[1]
    s = a1[:, 64:65]
    s = jnp.where(s > 0, s, 1.0)
    g1 = jnp.maximum(a1 / s + ba_ref[...], 0.0)
    g2 = jnp.maximum(a2 / s + bb_ref[...], 0.0)
    out_ref[...] = (jnp.dot(g1, wa_ref[...], preferred_element_type=jnp.float32)
                    + jnp.dot(g2, wb_ref[...], preferred_element_type=jnp.float32)
                    + fcb_ref[...])


def _row_spec(shape):
    nd = len(shape)
    if nd == 2:
        return pl.BlockSpec((_BLK, shape[1]), lambda i: (i, 0))
    return pl.BlockSpec((2, _BLK, shape[2]), lambda i: (0, i, 0))


def _full_spec(shape):
    nd = len(shape)
    return pl.BlockSpec(shape, lambda i: (0,) * nd)


def _tc_call(body, row_in, full_in, row_out_shapes, aux_out=True):
    in_specs = [_row_spec(a.shape) for a in row_in] + \
               [_full_spec(a.shape) for a in full_in]
    out_specs = [_row_spec(s) for s in row_out_shapes]
    out_shape = [jax.ShapeDtypeStruct(s, jnp.float32) for s in row_out_shapes]
    if aux_out:
        out_specs.append(pl.BlockSpec((16, _BLK), lambda i: (0, i)))
        out_shape.append(jax.ShapeDtypeStruct((16, NP), jnp.float32))
    return pl.pallas_call(
        body, grid=(_GRID,), in_specs=in_specs,
        out_specs=out_specs if len(out_specs) > 1 else out_specs[0],
        out_shape=out_shape if len(out_shape) > 1 else out_shape[0],
    )(*row_in, *full_in)


def _make_sc_edge(width, interpret=False):
    nvec = width // 16
    rows_per_sub = NP // 16

    def body(h_hbm, aux_hbm, src_hbm, dst_hbm, out_hbm,
             srcv, dstv, asv, adv, wfull, rows_a, rows_b, rows_c, acc,
             g_a, g_b, g_c, s_a, s_b, s_c):
        c = lax.axis_index("c")
        s = lax.axis_index("s")
        wid = s * 2 + c

        zero = jnp.zeros((16,), jnp.float32)

        def zr(e, carry):
            for j in range(nvec):
                rows_a[e, pl.ds(j * 16, 16)] = zero
                rows_c[e, pl.ds(j * 16, 16)] = zero
            return carry

        lax.fori_loop(0, CHUNK, zr, 0)
        for k in range(rows_per_sub // CHUNK):
            pltpu.sync_copy(rows_a,
                            acc.at[pl.ds(s * rows_per_sub + k * CHUNK, CHUNK)])

        pltpu.sync_copy(aux_hbm.at[0], asv)
        pltpu.sync_copy(aux_hbm.at[8], adv)
        pltpu.sync_copy(src_hbm.at[pl.ds(wid * ROWS_PER_TILE, ROWS_PER_TILE)],
                        srcv)
        pltpu.sync_copy(dst_hbm.at[pl.ds(wid * ROWS_PER_TILE, ROWS_PER_TILE)],
                        dstv)

        def wchunk(j, carry):
            base = j * CHUNK
            for g in range(CHUNK // 16):
                sv = srcv[j, pl.ds(g * 16, 16)]
                dv = dstv[j, pl.ds(g * 16, 16)]
                av = plsc.load_gather(asv, [sv])
                bv = plsc.load_gather(adv, [dv])
                ev = plsc.load_gather(asv, [dv])
                xl = av + bv
                l1 = jnp.where(xl >= 0, xl, 0.2 * xl)
                t2 = ev + bv
                cv = jnp.where(t2 >= 0, t2, 0.2 * t2)
                wfull[pl.ds(base + g * 16, 16)] = jnp.exp(l1 - cv)
            return carry

        lax.fori_loop(0, ROWS_PER_TILE, wchunk, 0)
        plsc.subcore_barrier()

        def gat(j, buf, sem):
            return pltpu.make_async_copy(h_hbm.at[srcv.at[j]], buf, sem)

        def scat_start(buf, j, sem):
            pltpu.async_copy(buf, acc.at[dstv.at[j]], sem, add=True)

        def scat_wait(buf, sem):
            pltpu.make_async_copy(buf, acc.at[dstv.at[0]], sem).wait()

        def scale(buf, j):
            def grp(g, carry2):
                wvec = wfull[pl.ds(j * CHUNK + g * 16, 16)]
                for u in range(16):
                    e = g * 16 + u
                    wsp = jnp.take_along_axis(
                        wvec, jnp.full((16,), u, jnp.int32), 0)
                    for jj in range(nvec):
                        buf[e, pl.ds(jj * 16, 16)] = \
                            buf[e, pl.ds(jj * 16, 16)] * wsp
                return carry2

            lax.fori_loop(0, CHUNK // 16, grp, 0)

        gat(0, rows_a, g_a).start()
        gat(1, rows_b, g_b).start()
        scat_start(rows_c, 0, s_c)

        def tri(k, carry):
            j0 = 3 * k
            last = ROWS_PER_TILE - 1
            scat_wait(rows_c, s_c)
            gat(j0 + 2, rows_c, g_c).start()
            gat(j0, rows_a, g_a).wait()
            scale(rows_a, j0)
            scat_start(rows_a, j0, s_a)
            gat(j0 + 1, rows_b, g_b).wait()
            scale(rows_b, j0 + 1)
            scat_start(rows_b, j0 + 1, s_b)
            scat_wait(rows_a, s_a)
            gat(jnp.minimum(j0 + 3, last), rows_a, g_a).start()
            gat(j0 + 2, rows_c, g_c).wait()
            scale(rows_c, j0 + 2)
            scat_start(rows_c, j0 + 2, s_c)
            scat_wait(rows_b, s_b)
            gat(jnp.minimum(j0 + 4, last), rows_b, g_b).start()
            return carry

        lax.fori_loop(0, ROWS_PER_TILE // 3, tri, 0)
        gat(ROWS_PER_TILE - 1, rows_a, g_a).wait()
        gat(ROWS_PER_TILE - 1, rows_b, g_b).wait()
        scat_wait(rows_c, s_c)
        plsc.subcore_barrier()

        off = s * rows_per_sub
        pltpu.sync_copy(acc.at[pl.ds(off, rows_per_sub)],
                        out_hbm.at[c].at[pl.ds(off, rows_per_sub)])

    mesh = plsc.VectorSubcoreMesh(core_axis_name="c", subcore_axis_name="s",
                                  num_cores=2, num_subcores=16)
    scratch = [
        pltpu.VMEM((ROWS_PER_TILE, CHUNK), jnp.int32),
        pltpu.VMEM((ROWS_PER_TILE, CHUNK), jnp.int32),
        pltpu.VMEM((NP,), jnp.float32),
        pltpu.VMEM((NP,), jnp.float32),
        pltpu.VMEM((ROWS_PER_TILE * CHUNK,), jnp.float32),
        pltpu.VMEM((CHUNK, width), jnp.float32),
        pltpu.VMEM((CHUNK, width), jnp.float32),
        pltpu.VMEM((CHUNK, width), jnp.float32),
        pltpu.VMEM_SHARED((NP, width), jnp.float32),
        pltpu.SemaphoreType.DMA,
        pltpu.SemaphoreType.DMA,
        pltpu.SemaphoreType.DMA,
        pltpu.SemaphoreType.DMA,
        pltpu.SemaphoreType.DMA,
        pltpu.SemaphoreType.DMA,
    ]
    return pl.kernel(body,
                     out_type=jax.ShapeDtypeStruct((2, NP, width), jnp.float32),
                     mesh=mesh, scratch_types=scratch, interpret=interpret,
                     compiler_params=pltpu.CompilerParams(
                         needs_layout_passes=False,
                         use_tc_tiling_on_sc=False))


_SC_EDGE = {w: _make_sc_edge(w) for w in (48, 80, 64)}


def kernel(x, edge_index, W1, att_src1, att_dst1, b1, W2, att_src2, att_dst2,
           b2, W3, att_src3, att_dst3, b3, W4, att_src4, att_dst4, b4,
           fc_W, fc_b):
    f32 = jnp.float32
    loops = jnp.arange(N_NODES, dtype=jnp.int32)
    pad_n = E_PAD - N_EDGES
    src = jnp.concatenate([edge_index[0], loops,
                           jnp.zeros((pad_n,), jnp.int32)]).reshape(-1, CHUNK)
    dst = jnp.concatenate([edge_index[1], loops,
                           jnp.full((pad_n,), N_NODES, jnp.int32)]).reshape(-1, CHUNK)
    x_pad = _pad2(x, NP, 128)

    def v2_of(v, width):
        out = jnp.zeros((16, width), f32)
        return out.at[0, :v[0].shape[0]].set(v[0]).at[8, :v[1].shape[0]].set(v[1])

    w1p = _pad2(W1, 128, 48)
    v2 = v2_of((W1 @ att_src1, W1 @ att_dst1), 128)
    h, aux = _tc_call(_tc_first_body, [x_pad], [w1p, v2],
                      [(NP, 48)])
    acc1 = _SC_EDGE[48](h, aux, src, dst)

    w2p = _pad2(W2, 48, 80)
    v2 = v2_of((W2 @ att_src2, W2 @ att_dst2), 48)
    bp = jnp.zeros((1, 48), f32).at[0, :32].set(b1)
    h, aux = _tc_call(_tc_mid1_body, [acc1], [w2p, v2, bp], [(NP, 80)])
    acc2 = _SC_EDGE[80](h, aux, src, dst)

    w3a = _pad2(W3[:, :64], 80, 80)
    w3b = _pad2(W3[:, 64:], 80, 64)
    v2 = v2_of((W3 @ att_src3, W3 @ att_dst3), 80)
    bp = jnp.zeros((1, 80), f32).at[0, :64].set(b2)
    h1, h2, aux = _tc_call(_tc_mid_s_body, [acc2], [w3a, w3b, v2, bp],
                           [(NP, 80), (NP, 64)])
    acc3a = _SC_EDGE[80](h1, aux, src, dst)
    acc3b = _SC_EDGE[64](h2, aux, src, dst)

    w4aa = _pad2(W4[:64, :64], 80, 80)
    w4ab = _pad2(W4[:64, 64:], 80, 64)
    w4ba = _pad2(W4[64:, :64], 64, 80)
    w4bb = W4[64:, 64:]
    v2a = v2_of((W4[:64] @ att_src4, W4[:64] @ att_dst4), 80)
    v2b = v2_of((W4[64:] @ att_src4, W4[64:] @ att_dst4), 64)
    bpa = jnp.zeros((1, 80), f32).at[0, :64].set(b3[:64])
    bpb = b3[64:].reshape(1, 64)
    h1, h2, aux = _tc_call(_tc_mid_ss_body, [acc3a, acc3b],
                           [w4aa, w4ab, w4ba, w4bb, v2a, v2b, bpa, bpb],
                           [(NP, 80), (NP, 64)])
    acc4a = _SC_EDGE[80](h1, aux, src, dst)
    acc4b = _SC_EDGE[64](h2, aux, src, dst)

    fca = _pad2(fc_W[:64], 80, 128)
    fcb_w = fc_W[64:]
    bpa = jnp.zeros((1, 80), f32).at[0, :64].set(b4[:64])
    bpb = b4[64:].reshape(1, 64)
    out = _tc_call(_tc_last_body, [acc4a, acc4b],
                   [fca, fcb_w, bpa, bpb, fc_b.reshape(1, 128)],
                   [(NP, 128)], aux_out=False)
    return out[:N_NODES]


# speedup vs baseline: 37.0243x; 1.1300x over previous
import jax
import jax.numpy as jnp
from jax import lax
from jax.experimental import pallas as pl
from jax.experimental.pallas import tpu as pltpu
from jax.experimental.pallas import tpu_sc as plsc

N_NODES = 10000
N_EDGES = 320000 + N_NODES
NP = 10240
CHUNK = 96
N_WORKERS = 32
ROWS_PER_TILE = 108
E_PAD = N_WORKERS * ROWS_PER_TILE * CHUNK
NEG_BIG = -1e30
_BLK = 256
_GRID = NP // _BLK


def _pad2(a, r, c):
    return jnp.pad(a, ((0, r - a.shape[0]), (0, c - a.shape[1])))


def _ones_col(h, col):
    ci = lax.broadcasted_iota(jnp.int32, h.shape, 1)
    return h + (ci == col).astype(jnp.float32)


def _aux(v2, g, aux_ref):
    aux_ref[...] = lax.dot_general(v2, g, (((1,), (1,)), ((), ())),
                                   preferred_element_type=jnp.float32)


def _norm(a, ic, b):
    s = a[:, ic:ic + 1]
    s = jnp.where(s > 0, s, 1.0)
    return jnp.maximum(a / s + b, 0.0)


def _tc_first_body(x_ref, w_ref, v2_ref, h_ref, aux_ref):
    g = x_ref[...]
    h = jnp.dot(g, w_ref[...], preferred_element_type=jnp.float32)
    h_ref[...] = _ones_col(h, 32)
    _aux(v2_ref[...], g, aux_ref)


def _tc_mid1_body(acc_ref, w_ref, v2_ref, b_ref, h_ref, aux_ref):
    g = _norm(acc_ref[0] + acc_ref[1], acc_ref.shape[2] - 16, b_ref[...])
    h = jnp.dot(g, w_ref[...], preferred_element_type=jnp.float32)
    h_ref[...] = _ones_col(h, w_ref.shape[1] - 16)
    _aux(v2_ref[...], g, aux_ref)


def _tc_mid_s_body(acc_ref, wa_ref, wb_ref, v2_ref, b_ref,
                   h1_ref, h2_ref, aux_ref):
    g = _norm(acc_ref[0] + acc_ref[1], acc_ref.shape[2] - 16, b_ref[...])
    h1 = jnp.dot(g, wa_ref[...], preferred_element_type=jnp.float32)
    h1_ref[...] = _ones_col(h1, 64)
    h2_ref[...] = jnp.dot(g, wb_ref[...], preferred_element_type=jnp.float32)
    _aux(v2_ref[...], g, aux_ref)


def _tc_mid_ss_body(a1_ref, a2_ref, waa_ref, wab_ref, wba_ref, wbb_ref,
                    v2a_ref, v2b_ref, ba_ref, bb_ref, h1_ref, h2_ref, aux_ref):
    a1 = a1_ref[0] + a1_ref[1]
    a2 = a2_ref[0] + a2_ref[1]
    s = a1[:, 64:65]
    s = jnp.where(s > 0, s, 1.0)
    g1 = jnp.maximum(a1 / s + ba_ref[...], 0.0)
    g2 = jnp.maximum(a2 / s + bb_ref[...], 0.0)
    h1 = (jnp.dot(g1, waa_ref[...], preferred_element_type=jnp.float32)
          + jnp.dot(g2, wba_ref[...], preferred_element_type=jnp.float32))
    h1_ref[...] = _ones_col(h1, 64)
    h2_ref[...] = (jnp.dot(g1, wab_ref[...], preferred_element_type=jnp.float32)
                   + jnp.dot(g2, wbb_ref[...], preferred_element_type=jnp.float32))
    aux_ref[...] = (
        lax.dot_general(v2a_ref[...], g1, (((1,), (1,)), ((), ())),
                        preferred_element_type=jnp.float32)
        + lax.dot_general(v2b_ref[...], g2, (((1,), (1,)), ((), ())),
                          preferred_element_type=jnp.float32))


def _tc_last_body(a1_ref, a2_ref, wa_ref, wb_ref, ba_ref, bb_ref, fcb_ref,
                  out_ref):
    a1 = a1_ref[0] + a1_ref[1]
    a2 = a2_ref[0] + a2_ref[1]
    s = a1[:, 64:65]
    s = jnp.where(s > 0, s, 1.0)
    g1 = jnp.maximum(a1 / s + ba_ref[...], 0.0)
    g2 = jnp.maximum(a2 / s + bb_ref[...], 0.0)
    out_ref[...] = (jnp.dot(g1, wa_ref[...], preferred_element_type=jnp.float32)
                    + jnp.dot(g2, wb_ref[...], preferred_element_type=jnp.float32)
                    + fcb_ref[...])


def _row_spec(shape):
    nd = len(shape)
    if nd == 2:
        return pl.BlockSpec((_BLK, shape[1]), lambda i: (i, 0))
    return pl.BlockSpec((2, _BLK, shape[2]), lambda i: (0, i, 0))


def _full_spec(shape):
    nd = len(shape)
    return pl.BlockSpec(shape, lambda i: (0,) * nd)


def _tc_call(body, row_in, full_in, row_out_shapes, aux_out=True):
    in_specs = [_row_spec(a.shape) for a in row_in] + \
               [_full_spec(a.shape) for a in full_in]
    out_specs = [_row_spec(s) for s in row_out_shapes]
    out_shape = [jax.ShapeDtypeStruct(s, jnp.float32) for s in row_out_shapes]
    if aux_out:
        out_specs.append(pl.BlockSpec((16, _BLK), lambda i: (0, i)))
        out_shape.append(jax.ShapeDtypeStruct((16, NP), jnp.float32))
    return pl.pallas_call(
        body, grid=(_GRID,), in_specs=in_specs,
        out_specs=out_specs if len(out_specs) > 1 else out_specs[0],
        out_shape=out_shape if len(out_shape) > 1 else out_shape[0],
    )(*row_in, *full_in)


def _make_sc_edge(width, nphase=1, interpret=False):
    nvec = width // 16
    rows_per_sub = NP // 16

    def body(*refs):
        hs = refs[:nphase]
        aux_hbm, src_hbm, dst_hbm = refs[nphase:nphase + 3]
        outs = refs[nphase + 3:nphase + 3 + nphase]
        (srcv, dstv, asv, adv, wfull, rows_a, rows_b, rows_c, acc,
         g_a, g_b, g_c, s_a, s_b, s_c) = refs[nphase + 3 + nphase:]
        c = lax.axis_index("c")
        s = lax.axis_index("s")
        wid = s * 2 + c

        zero = jnp.zeros((16,), jnp.float32)

        def zr(e, carry):
            for j in range(nvec):
                rows_a[e, pl.ds(j * 16, 16)] = zero
                rows_c[e, pl.ds(j * 16, 16)] = zero
            return carry

        def zero_acc():
            lax.fori_loop(0, CHUNK, zr, 0)
            base = s * rows_per_sub
            nfull = rows_per_sub // CHUNK
            rem = rows_per_sub % CHUNK
            for k in range(nfull):
                pltpu.sync_copy(rows_a, acc.at[pl.ds(base + k * CHUNK, CHUNK)])
            if rem:
                pltpu.sync_copy(rows_a.at[pl.ds(0, rem)],
                                acc.at[pl.ds(base + nfull * CHUNK, rem)])

        pltpu.sync_copy(aux_hbm.at[0], asv)
        pltpu.sync_copy(aux_hbm.at[8], adv)
        pltpu.sync_copy(src_hbm.at[pl.ds(wid * ROWS_PER_TILE, ROWS_PER_TILE)],
                        srcv)
        pltpu.sync_copy(dst_hbm.at[pl.ds(wid * ROWS_PER_TILE, ROWS_PER_TILE)],
                        dstv)

        def wchunk(j, carry):
            base = j * CHUNK
            for g in range(CHUNK // 16):
                sv = srcv[j, pl.ds(g * 16, 16)]
                dv = dstv[j, pl.ds(g * 16, 16)]
                av = plsc.load_gather(asv, [sv])
                bv = plsc.load_gather(adv, [dv])
                ev = plsc.load_gather(asv, [dv])
                xl = av + bv
                l1 = jnp.where(xl >= 0, xl, 0.2 * xl)
                t2 = ev + bv
                cv = jnp.where(t2 >= 0, t2, 0.2 * t2)
                wfull[pl.ds(base + g * 16, 16)] = jnp.exp(l1 - cv)
            return carry

        lax.fori_loop(0, ROWS_PER_TILE, wchunk, 0)

        def scat_start(buf, j, sem):
            pltpu.async_copy(buf, acc.at[dstv.at[j]], sem, add=True)

        def scat_wait(buf, sem):
            pltpu.make_async_copy(buf, acc.at[dstv.at[0]], sem).wait()

        def scale(buf, j):
            def grp(g, carry2):
                wvec = wfull[pl.ds(j * CHUNK + g * 16, 16)]
                for u in range(16):
                    e = g * 16 + u
                    wsp = jnp.take_along_axis(
                        wvec, jnp.full((16,), u, jnp.int32), 0)
                    for jj in range(nvec):
                        buf[e, pl.ds(jj * 16, 16)] = \
                            buf[e, pl.ds(jj * 16, 16)] * wsp
                return carry2

            lax.fori_loop(0, CHUNK // 16, grp, 0)

        def pipeline(h_hbm, out_hbm):
            def gat(j, buf, sem):
                return pltpu.make_async_copy(h_hbm.at[srcv.at[j]], buf, sem)

            zero_acc()
            plsc.subcore_barrier()
            gat(0, rows_a, g_a).start()
            gat(1, rows_b, g_b).start()
            scat_start(rows_c, 0, s_c)

            def tri(k, carry):
                j0 = 3 * k
                last = ROWS_PER_TILE - 1
                scat_wait(rows_c, s_c)
                gat(j0 + 2, rows_c, g_c).start()
                gat(j0, rows_a, g_a).wait()
                scale(rows_a, j0)
                scat_start(rows_a, j0, s_a)
                gat(j0 + 1, rows_b, g_b).wait()
                scale(rows_b, j0 + 1)
                scat_start(rows_b, j0 + 1, s_b)
                scat_wait(rows_a, s_a)
                gat(jnp.minimum(j0 + 3, last), rows_a, g_a).start()
                gat(j0 + 2, rows_c, g_c).wait()
                scale(rows_c, j0 + 2)
                scat_start(rows_c, j0 + 2, s_c)
                scat_wait(rows_b, s_b)
                gat(jnp.minimum(j0 + 4, last), rows_b, g_b).start()
                return carry

            lax.fori_loop(0, ROWS_PER_TILE // 3, tri, 0)
            gat(ROWS_PER_TILE - 1, rows_a, g_a).wait()
            gat(ROWS_PER_TILE - 1, rows_b, g_b).wait()
            scat_wait(rows_c, s_c)
            plsc.subcore_barrier()
            off = s * rows_per_sub
            pltpu.sync_copy(acc.at[pl.ds(off, rows_per_sub)],
                            out_hbm.at[c].at[pl.ds(off, rows_per_sub)])

        for p in range(nphase):
            pipeline(hs[p], outs[p])

    mesh = plsc.VectorSubcoreMesh(core_axis_name="c", subcore_axis_name="s",
                                  num_cores=2, num_subcores=16)
    scratch = [
        pltpu.VMEM((ROWS_PER_TILE, CHUNK), jnp.int32),
        pltpu.VMEM((ROWS_PER_TILE, CHUNK), jnp.int32),
        pltpu.VMEM((NP,), jnp.float32),
        pltpu.VMEM((NP,), jnp.float32),
        pltpu.VMEM((ROWS_PER_TILE * CHUNK,), jnp.float32),
        pltpu.VMEM((CHUNK, width), jnp.float32),
        pltpu.VMEM((CHUNK, width), jnp.float32),
        pltpu.VMEM((CHUNK, width), jnp.float32),
        pltpu.VMEM_SHARED((NP, width), jnp.float32),
        pltpu.SemaphoreType.DMA,
        pltpu.SemaphoreType.DMA,
        pltpu.SemaphoreType.DMA,
        pltpu.SemaphoreType.DMA,
        pltpu.SemaphoreType.DMA,
        pltpu.SemaphoreType.DMA,
    ]
    out_type = [jax.ShapeDtypeStruct((2, NP, width), jnp.float32)
                for _ in range(nphase)]
    return pl.kernel(body,
                     out_type=out_type[0] if nphase == 1 else tuple(out_type),
                     mesh=mesh, scratch_types=scratch, interpret=interpret,
                     compiler_params=pltpu.CompilerParams(
                         needs_layout_passes=False,
                         use_tc_tiling_on_sc=False))


_SC_EDGE = {48: _make_sc_edge(48), 80: _make_sc_edge(80),
            "dual80": _make_sc_edge(80, nphase=2)}


def kernel(x, edge_index, W1, att_src1, att_dst1, b1, W2, att_src2, att_dst2,
           b2, W3, att_src3, att_dst3, b3, W4, att_src4, att_dst4, b4,
           fc_W, fc_b):
    f32 = jnp.float32
    loops = jnp.arange(N_NODES, dtype=jnp.int32)
    pad_n = E_PAD - N_EDGES
    src = jnp.concatenate([edge_index[0], loops,
                           jnp.zeros((pad_n,), jnp.int32)]).reshape(-1, CHUNK)
    dst = jnp.concatenate([edge_index[1], loops,
                           jnp.full((pad_n,), N_NODES, jnp.int32)]).reshape(-1, CHUNK)
    x_pad = _pad2(x, NP, 128)

    def v2_of(v, width):
        out = jnp.zeros((16, width), f32)
        return out.at[0, :v[0].shape[0]].set(v[0]).at[8, :v[1].shape[0]].set(v[1])

    w1p = _pad2(W1, 128, 48)
    v2 = v2_of((W1 @ att_src1, W1 @ att_dst1), 128)
    h, aux = _tc_call(_tc_first_body, [x_pad], [w1p, v2],
                      [(NP, 48)])
    acc1 = _SC_EDGE[48](h, aux, src, dst)

    w2p = _pad2(W2, 48, 80)
    v2 = v2_of((W2 @ att_src2, W2 @ att_dst2), 48)
    bp = jnp.zeros((1, 48), f32).at[0, :32].set(b1)
    h, aux = _tc_call(_tc_mid1_body, [acc1], [w2p, v2, bp], [(NP, 80)])
    acc2 = _SC_EDGE[80](h, aux, src, dst)

    w3a = _pad2(W3[:, :64], 80, 80)
    w3b = _pad2(W3[:, 64:], 80, 80)
    v2 = v2_of((W3 @ att_src3, W3 @ att_dst3), 80)
    bp = jnp.zeros((1, 80), f32).at[0, :64].set(b2)
    h1, h2, aux = _tc_call(_tc_mid_s_body, [acc2], [w3a, w3b, v2, bp],
                           [(NP, 80), (NP, 80)])
    acc3a, acc3b = _SC_EDGE["dual80"](h1, h2, aux, src, dst)

    w4aa = _pad2(W4[:64, :64], 80, 80)
    w4ab = _pad2(W4[:64, 64:], 80, 80)
    w4ba = _pad2(W4[64:, :64], 80, 80)
    w4bb = _pad2(W4[64:, 64:], 80, 80)
    v2a = v2_of((W4[:64] @ att_src4, W4[:64] @ att_dst4), 80)
    v2b = v2_of((W4[64:] @ att_src4, W4[64:] @ att_dst4), 80)
    bpa = jnp.zeros((1, 80), f32).at[0, :64].set(b3[:64])
    bpb = jnp.zeros((1, 80), f32).at[0, :64].set(b3[64:])
    h1, h2, aux = _tc_call(_tc_mid_ss_body, [acc3a, acc3b],
                           [w4aa, w4ab, w4ba, w4bb, v2a, v2b, bpa, bpb],
                           [(NP, 80), (NP, 80)])
    acc4a, acc4b = _SC_EDGE["dual80"](h1, h2, aux, src, dst)

    fca = _pad2(fc_W[:64], 80, 128)
    fcb_w = _pad2(fc_W[64:], 80, 128)
    bpa = jnp.zeros((1, 80), f32).at[0, :64].set(b4[:64])
    bpb = jnp.zeros((1, 80), f32).at[0, :64].set(b4[64:])
    out = _tc_call(_tc_last_body, [acc4a, acc4b],
                   [fca, fcb_w, bpa, bpb, fc_b.reshape(1, 128)],
                   [(NP, 128)], aux_out=False)
    return out[:N_NODES]


# speedup vs baseline: 38.8095x; 1.0482x over previous
import jax
import jax.numpy as jnp
from jax import lax
from jax.experimental import pallas as pl
from jax.experimental.pallas import tpu as pltpu
from jax.experimental.pallas import tpu_sc as plsc

N_NODES = 10000
N_EDGES = 320000 + N_NODES
NP = 10240
CHUNK = 128
N_WORKERS = 32
ROWS_PER_TILE = 81
E_PAD = N_WORKERS * ROWS_PER_TILE * CHUNK
NEG_BIG = -1e30
_BLK = 256
_GRID = NP // _BLK


def _pad2(a, r, c):
    return jnp.pad(a, ((0, r - a.shape[0]), (0, c - a.shape[1])))


def _ones_col(h, col):
    ci = lax.broadcasted_iota(jnp.int32, h.shape, 1)
    return h + (ci == col).astype(jnp.float32)


def _aux(v2, g, aux_ref):
    aux_ref[...] = lax.dot_general(v2, g, (((1,), (1,)), ((), ())),
                                   preferred_element_type=jnp.float32)


def _norm(a, ic, b):
    s = a[:, ic:ic + 1]
    s = jnp.where(s > 0, s, 1.0)
    return jnp.maximum(a / s + b, 0.0)


def _tc_first_body(x_ref, w_ref, v2_ref, h_ref, aux_ref):
    g = x_ref[...]
    h = jnp.dot(g, w_ref[...], preferred_element_type=jnp.float32)
    h_ref[...] = _ones_col(h, 32)
    _aux(v2_ref[...], g, aux_ref)


def _tc_mid1_body(acc_ref, w_ref, v2_ref, b_ref, h_ref, aux_ref):
    g = _norm(acc_ref[0] + acc_ref[1], acc_ref.shape[2] - 16, b_ref[...])
    h = jnp.dot(g, w_ref[...], preferred_element_type=jnp.float32)
    h_ref[...] = _ones_col(h, w_ref.shape[1] - 16)
    _aux(v2_ref[...], g, aux_ref)


def _tc_mid_s_body(acc_ref, wa_ref, wb_ref, v2_ref, b_ref,
                   h1_ref, h2_ref, aux_ref):
    g = _norm(acc_ref[0] + acc_ref[1], acc_ref.shape[2] - 16, b_ref[...])
    h1 = jnp.dot(g, wa_ref[...], preferred_element_type=jnp.float32)
    h1_ref[...] = _ones_col(h1, 64)
    h2_ref[...] = jnp.dot(g, wb_ref[...], preferred_element_type=jnp.float32)
    _aux(v2_ref[...], g, aux_ref)


def _tc_mid_ss_body(a1_ref, a2_ref, waa_ref, wab_ref, wba_ref, wbb_ref,
                    v2a_ref, v2b_ref, ba_ref, bb_ref, h1_ref, h2_ref, aux_ref):
    a1 = a1_ref[0] + a1_ref[1]
    a2 = a2_ref[0] + a2_ref[1]
    s = a1[:, 64:65]
    s = jnp.where(s > 0, s, 1.0)
    g1 = jnp.maximum(a1 / s + ba_ref[...], 0.0)
    g2 = jnp.maximum(a2 / s + bb_ref[...], 0.0)
    h1 = (jnp.dot(g1, waa_ref[...], preferred_element_type=jnp.float32)
          + jnp.dot(g2, wba_ref[...], preferred_element_type=jnp.float32))
    h1_ref[...] = _ones_col(h1, 64)
    h2_ref[...] = (jnp.dot(g1, wab_ref[...], preferred_element_type=jnp.float32)
                   + jnp.dot(g2, wbb_ref[...], preferred_element_type=jnp.float32))
    aux_ref[...] = (
        lax.dot_general(v2a_ref[...], g1, (((1,), (1,)), ((), ())),
                        preferred_element_type=jnp.float32)
        + lax.dot_general(v2b_ref[...], g2, (((1,), (1,)), ((), ())),
                          preferred_element_type=jnp.float32))


def _tc_last_body(a1_ref, a2_ref, wa_ref, wb_ref, ba_ref, bb_ref, fcb_ref,
                  out_ref):
    a1 = a1_ref[0] + a1_ref[1]
    a2 = a2_ref[0] + a2_ref[1]
    s = a1[:, 64:65]
    s = jnp.where(s > 0, s, 1.0)
    g1 = jnp.maximum(a1 / s + ba_ref[...], 0.0)
    g2 = jnp.maximum(a2 / s + bb_ref[...], 0.0)
    out_ref[...] = (jnp.dot(g1, wa_ref[...], preferred_element_type=jnp.float32)
                    + jnp.dot(g2, wb_ref[...], preferred_element_type=jnp.float32)
                    + fcb_ref[...])


def _row_spec(shape):
    nd = len(shape)
    if nd == 2:
        return pl.BlockSpec((_BLK, shape[1]), lambda i: (i, 0))
    return pl.BlockSpec((2, _BLK, shape[2]), lambda i: (0, i, 0))


def _full_spec(shape):
    nd = len(shape)
    return pl.BlockSpec(shape, lambda i: (0,) * nd)


def _tc_call(body, row_in, full_in, row_out_shapes, aux_out=True):
    in_specs = [_row_spec(a.shape) for a in row_in] + \
               [_full_spec(a.shape) for a in full_in]
    out_specs = [_row_spec(s) for s in row_out_shapes]
    out_shape = [jax.ShapeDtypeStruct(s, jnp.float32) for s in row_out_shapes]
    if aux_out:
        out_specs.append(pl.BlockSpec((16, _BLK), lambda i: (0, i)))
        out_shape.append(jax.ShapeDtypeStruct((16, NP), jnp.float32))
    return pl.pallas_call(
        body, grid=(_GRID,), in_specs=in_specs,
        out_specs=out_specs if len(out_specs) > 1 else out_specs[0],
        out_shape=out_shape if len(out_shape) > 1 else out_shape[0],
    )(*row_in, *full_in)


def _make_sc_edge(width, nphase=1, interpret=False):
    nvec = width // 16
    rows_per_sub = NP // 16

    def body(*refs):
        hs = refs[:nphase]
        aux_hbm, src_hbm, dst_hbm = refs[nphase:nphase + 3]
        outs = refs[nphase + 3:nphase + 3 + nphase]
        (srcv, dstv, asv, adv, wbuf, rows_a, rows_b, rows_c, acc,
         g_a, g_b, g_c, s_a, s_b, s_c) = refs[nphase + 3 + nphase:]
        c = lax.axis_index("c")
        s = lax.axis_index("s")
        wid = s * 2 + c

        zero = jnp.zeros((16,), jnp.float32)

        def zr(e, carry):
            for j in range(nvec):
                rows_a[e, pl.ds(j * 16, 16)] = zero
                rows_c[e, pl.ds(j * 16, 16)] = zero
            return carry

        def zero_acc():
            lax.fori_loop(0, CHUNK, zr, 0)
            base = s * rows_per_sub
            nfull = rows_per_sub // CHUNK
            rem = rows_per_sub % CHUNK
            for k in range(nfull):
                pltpu.sync_copy(rows_a, acc.at[pl.ds(base + k * CHUNK, CHUNK)])
            if rem:
                pltpu.sync_copy(rows_a.at[pl.ds(0, rem)],
                                acc.at[pl.ds(base + nfull * CHUNK, rem)])

        pltpu.sync_copy(aux_hbm.at[0], asv)
        pltpu.sync_copy(aux_hbm.at[8], adv)
        pltpu.sync_copy(src_hbm.at[pl.ds(wid * ROWS_PER_TILE, ROWS_PER_TILE)],
                        srcv)
        pltpu.sync_copy(dst_hbm.at[pl.ds(wid * ROWS_PER_TILE, ROWS_PER_TILE)],
                        dstv)

        def compute_w(j):
            for g in range(CHUNK // 16):
                sv = srcv[j, pl.ds(g * 16, 16)]
                dv = dstv[j, pl.ds(g * 16, 16)]
                av = plsc.load_gather(asv, [sv])
                bv = plsc.load_gather(adv, [dv])
                ev = plsc.load_gather(asv, [dv])
                xl = av + bv
                l1 = jnp.where(xl >= 0, xl, 0.2 * xl)
                t2 = ev + bv
                cv = jnp.where(t2 >= 0, t2, 0.2 * t2)
                wbuf[pl.ds(g * 16, 16)] = jnp.exp(l1 - cv)

        def scat_start(buf, j, sem):
            pltpu.async_copy(buf, acc.at[dstv.at[j]], sem, add=True)

        def scat_wait(buf, sem):
            pltpu.make_async_copy(buf, acc.at[dstv.at[0]], sem).wait()

        def scale(buf, j):
            del j
            def grp(g, carry2):
                wvec = wbuf[pl.ds(g * 16, 16)]
                for u in range(16):
                    e = g * 16 + u
                    wsp = jnp.take_along_axis(
                        wvec, jnp.full((16,), u, jnp.int32), 0)
                    for jj in range(nvec):
                        buf[e, pl.ds(jj * 16, 16)] = \
                            buf[e, pl.ds(jj * 16, 16)] * wsp
                return carry2

            lax.fori_loop(0, CHUNK // 16, grp, 0)

        def pipeline(h_hbm, out_hbm):
            def gat(j, buf, sem):
                return pltpu.make_async_copy(h_hbm.at[srcv.at[j]], buf, sem)

            zero_acc()
            plsc.subcore_barrier()
            gat(0, rows_a, g_a).start()
            gat(1, rows_b, g_b).start()
            scat_start(rows_c, 0, s_c)

            def tri(k, carry):
                j0 = 3 * k
                last = ROWS_PER_TILE - 1
                scat_wait(rows_c, s_c)
                gat(j0 + 2, rows_c, g_c).start()
                compute_w(j0)
                gat(j0, rows_a, g_a).wait()
                scale(rows_a, j0)
                scat_start(rows_a, j0, s_a)
                compute_w(j0 + 1)
                gat(j0 + 1, rows_b, g_b).wait()
                scale(rows_b, j0 + 1)
                scat_start(rows_b, j0 + 1, s_b)
                scat_wait(rows_a, s_a)
                gat(jnp.minimum(j0 + 3, last), rows_a, g_a).start()
                compute_w(j0 + 2)
                gat(j0 + 2, rows_c, g_c).wait()
                scale(rows_c, j0 + 2)
                scat_start(rows_c, j0 + 2, s_c)
                scat_wait(rows_b, s_b)
                gat(jnp.minimum(j0 + 4, last), rows_b, g_b).start()
                return carry

            lax.fori_loop(0, ROWS_PER_TILE // 3, tri, 0)
            gat(ROWS_PER_TILE - 1, rows_a, g_a).wait()
            gat(ROWS_PER_TILE - 1, rows_b, g_b).wait()
            scat_wait(rows_c, s_c)
            plsc.subcore_barrier()
            off = s * rows_per_sub
            pltpu.sync_copy(acc.at[pl.ds(off, rows_per_sub)],
                            out_hbm.at[c].at[pl.ds(off, rows_per_sub)])

        for p in range(nphase):
            pipeline(hs[p], outs[p])

    mesh = plsc.VectorSubcoreMesh(core_axis_name="c", subcore_axis_name="s",
                                  num_cores=2, num_subcores=16)
    scratch = [
        pltpu.VMEM((ROWS_PER_TILE, CHUNK), jnp.int32),
        pltpu.VMEM((ROWS_PER_TILE, CHUNK), jnp.int32),
        pltpu.VMEM((NP,), jnp.float32),
        pltpu.VMEM((NP,), jnp.float32),
        pltpu.VMEM((CHUNK,), jnp.float32),
        pltpu.VMEM((CHUNK, width), jnp.float32),
        pltpu.VMEM((CHUNK, width), jnp.float32),
        pltpu.VMEM((CHUNK, width), jnp.float32),
        pltpu.VMEM_SHARED((NP, width), jnp.float32),
        pltpu.SemaphoreType.DMA,
        pltpu.SemaphoreType.DMA,
        pltpu.SemaphoreType.DMA,
        pltpu.SemaphoreType.DMA,
        pltpu.SemaphoreType.DMA,
        pltpu.SemaphoreType.DMA,
    ]
    out_type = [jax.ShapeDtypeStruct((2, NP, width), jnp.float32)
                for _ in range(nphase)]
    return pl.kernel(body,
                     out_type=out_type[0] if nphase == 1 else tuple(out_type),
                     mesh=mesh, scratch_types=scratch, interpret=interpret,
                     compiler_params=pltpu.CompilerParams(
                         needs_layout_passes=False,
                         use_tc_tiling_on_sc=False))


_SC_EDGE = {48: _make_sc_edge(48), 80: _make_sc_edge(80),
            "dual80": _make_sc_edge(80, nphase=2)}


def kernel(x, edge_index, W1, att_src1, att_dst1, b1, W2, att_src2, att_dst2,
           b2, W3, att_src3, att_dst3, b3, W4, att_src4, att_dst4, b4,
           fc_W, fc_b):
    f32 = jnp.float32
    loops = jnp.arange(N_NODES, dtype=jnp.int32)
    pad_n = E_PAD - N_EDGES
    src = jnp.concatenate([edge_index[0], loops,
                           jnp.zeros((pad_n,), jnp.int32)]).reshape(-1, CHUNK)
    dst = jnp.concatenate([edge_index[1], loops,
                           jnp.full((pad_n,), N_NODES, jnp.int32)]).reshape(-1, CHUNK)
    x_pad = _pad2(x, NP, 128)

    def v2_of(v, width):
        out = jnp.zeros((16, width), f32)
        return out.at[0, :v[0].shape[0]].set(v[0]).at[8, :v[1].shape[0]].set(v[1])

    w1p = _pad2(W1, 128, 48)
    v2 = v2_of((W1 @ att_src1, W1 @ att_dst1), 128)
    h, aux = _tc_call(_tc_first_body, [x_pad], [w1p, v2],
                      [(NP, 48)])
    acc1 = _SC_EDGE[48](h, aux, src, dst)

    w2p = _pad2(W2, 48, 80)
    v2 = v2_of((W2 @ att_src2, W2 @ att_dst2), 48)
    bp = jnp.zeros((1, 48), f32).at[0, :32].set(b1)
    h, aux = _tc_call(_tc_mid1_body, [acc1], [w2p, v2, bp], [(NP, 80)])
    acc2 = _SC_EDGE[80](h, aux, src, dst)

    w3a = _pad2(W3[:, :64], 80, 80)
    w3b = _pad2(W3[:, 64:], 80, 80)
    v2 = v2_of((W3 @ att_src3, W3 @ att_dst3), 80)
    bp = jnp.zeros((1, 80), f32).at[0, :64].set(b2)
    h1, h2, aux = _tc_call(_tc_mid_s_body, [acc2], [w3a, w3b, v2, bp],
                           [(NP, 80), (NP, 80)])
    acc3a, acc3b = _SC_EDGE["dual80"](h1, h2, aux, src, dst)

    w4aa = _pad2(W4[:64, :64], 80, 80)
    w4ab = _pad2(W4[:64, 64:], 80, 80)
    w4ba = _pad2(W4[64:, :64], 80, 80)
    w4bb = _pad2(W4[64:, 64:], 80, 80)
    v2a = v2_of((W4[:64] @ att_src4, W4[:64] @ att_dst4), 80)
    v2b = v2_of((W4[64:] @ att_src4, W4[64:] @ att_dst4), 80)
    bpa = jnp.zeros((1, 80), f32).at[0, :64].set(b3[:64])
    bpb = jnp.zeros((1, 80), f32).at[0, :64].set(b3[64:])
    h1, h2, aux = _tc_call(_tc_mid_ss_body, [acc3a, acc3b],
                           [w4aa, w4ab, w4ba, w4bb, v2a, v2b, bpa, bpb],
                           [(NP, 80), (NP, 80)])
    acc4a, acc4b = _SC_EDGE["dual80"](h1, h2, aux, src, dst)

    fca = _pad2(fc_W[:64], 80, 128)
    fcb_w = _pad2(fc_W[64:], 80, 128)
    bpa = jnp.zeros((1, 80), f32).at[0, :64].set(b4[:64])
    bpb = jnp.zeros((1, 80), f32).at[0, :64].set(b4[64:])
    out = _tc_call(_tc_last_body, [acc4a, acc4b],
                   [fca, fcb_w, bpa, bpb, fc_b.reshape(1, 128)],
                   [(NP, 128)], aux_out=False)
    return out[:N_NODES]
